# baseline (speedup 1.0000x reference)
import functools

import jax
import jax.numpy as jnp
from jax.experimental import pallas as pl
from jax.experimental.pallas import tpu as pltpu


def _fused_body(LA_ref, LB_ref, F_ref, Wc_ref, bc_ref, out_ref, *, bm):
    i = pl.program_id(0)

    def half(L_ref, blk):
        x = jnp.dot(L_ref[...], F_ref[...], preferred_element_type=jnp.float32)
        f_row = F_ref[pl.ds(blk * bm, bm), :]
        lhs = jnp.concatenate([f_row + x, x * f_row], axis=1)
        return (
            jnp.dot(lhs, Wc_ref[...], preferred_element_type=jnp.float32)
            + bc_ref[...]
        )

    out_ref[0:bm, :] = half(LA_ref, 2 * i)
    out_ref[bm:2 * bm, :] = half(LB_ref, 2 * i + 1)


def kernel(lap_matrix, eye_matrix, features, W1, b1, W2, b2):
    n, d = features.shape
    bm = 200

    Wc = jnp.concatenate([W1.T, W2.T], axis=0)
    bc = (b1 + b2).reshape(1, d)

    body = functools.partial(_fused_body, bm=bm)
    return pl.pallas_call(
        body,
        grid=(n // (2 * bm),),
        in_specs=[
            pl.BlockSpec((bm, n), lambda i: (2 * i, 0)),
            pl.BlockSpec((bm, n), lambda i: (2 * i + 1, 0)),
            pl.BlockSpec((n, d), lambda i: (0, 0)),
            pl.BlockSpec((2 * d, d), lambda i: (0, 0)),
            pl.BlockSpec((1, d), lambda i: (0, 0)),
        ],
        out_specs=pl.BlockSpec((2 * bm, d), lambda i: (i, 0)),
        out_shape=jax.ShapeDtypeStruct((n, d), jnp.float32),
        compiler_params=pltpu.CompilerParams(
            dimension_semantics=("parallel",),
        ),
    )(lap_matrix, lap_matrix, features, Wc, bc)


# speedup vs baseline: 1.0656x; 1.0656x over previous
import functools

import jax
import jax.numpy as jnp
from jax.experimental import pallas as pl
from jax.experimental.pallas import tpu as pltpu


def _fused_body(L_ref, F_ref, Wc_ref, bc_ref, out_ref, *, bm):
    i = pl.program_id(0)
    x = jnp.dot(
        L_ref[...].astype(jnp.bfloat16),
        F_ref[...].astype(jnp.bfloat16),
        preferred_element_type=jnp.float32,
    )
    f_row = F_ref[pl.ds(i * bm, bm), :]
    lhs = jnp.concatenate([f_row + x, x * f_row], axis=1)
    out_ref[...] = (
        jnp.dot(lhs, Wc_ref[...], preferred_element_type=jnp.float32)
        + bc_ref[...]
    )


def kernel(lap_matrix, eye_matrix, features, W1, b1, W2, b2):
    n, d = features.shape
    bm = 400

    Wc = jnp.concatenate([W1.T, W2.T], axis=0)
    bc = (b1 + b2).reshape(1, d)

    body = functools.partial(_fused_body, bm=bm)
    return pl.pallas_call(
        body,
        grid=(pl.cdiv(n, bm),),
        in_specs=[
            pl.BlockSpec((bm, n), lambda i: (i, 0)),
            pl.BlockSpec((n, d), lambda i: (0, 0)),
            pl.BlockSpec((2 * d, d), lambda i: (0, 0)),
            pl.BlockSpec((1, d), lambda i: (0, 0)),
        ],
        out_specs=pl.BlockSpec((bm, d), lambda i: (i, 0)),
        out_shape=jax.ShapeDtypeStruct((n, d), jnp.float32),
        compiler_params=pltpu.CompilerParams(
            dimension_semantics=("parallel",),
        ),
    )(lap_matrix, features, Wc, bc)
